# baseline (device time: 36763 ns/iter reference)
import jax
import jax.numpy as jnp
from jax import lax
from jax.experimental import pallas as pl
from jax.experimental.pallas import tpu as pltpu

M = 2048
M_HALF = 1024
N_HALF = 512
TILES = [64] * 15 + [32, 32]
OFFS = [sum(TILES[:i]) for i in range(len(TILES))]
T = len(TILES)


def kernel(x):
    def body(x_ref, out_ref, local_ref, recv_y_ref, recv_x_ref,
             local_sem, send_sems_y, recv_sems_y, send_sems_x, recv_sems_x):
        my_x = lax.axis_index("x")
        my_y = lax.axis_index("y")
        y_nbr = (my_x, 1 - my_y)
        x_nbr = (1 - my_x, my_y)

        row_me = my_x * M_HALF
        row_other = (1 - my_x) * M_HALF
        col_me = my_y * N_HALF
        col_nbr = (1 - my_y) * N_HALF

        local_copy = pltpu.make_async_copy(
            x_ref.at[0, :, pl.ds(col_me, N_HALF)], local_ref, local_sem,
        )
        local_copy.start()

        barrier_sem = pltpu.get_barrier_semaphore()
        for nbr in (y_nbr, x_nbr):
            pl.semaphore_signal(
                barrier_sem, inc=1,
                device_id=nbr, device_id_type=pl.DeviceIdType.MESH,
            )
        pl.semaphore_wait(barrier_sem, 2)

        y_rdmas = []
        for t, (off, sz) in enumerate(zip(OFFS, TILES)):
            rdma = pltpu.make_async_remote_copy(
                src_ref=x_ref.at[0, pl.ds(row_me + off, sz),
                                 pl.ds(col_nbr, N_HALF)],
                dst_ref=recv_y_ref.at[pl.ds(off, sz)],
                send_sem=send_sems_y.at[t],
                recv_sem=recv_sems_y.at[t],
                device_id=y_nbr,
                device_id_type=pl.DeviceIdType.MESH,
            )
            rdma.start()
            y_rdmas.append(rdma)

        local_copy.wait()

        x_rdmas = []
        for t, (off, sz) in enumerate(zip(OFFS, TILES)):
            y_rdmas[t].wait_recv()
            rdma = pltpu.make_async_remote_copy(
                src_ref=recv_y_ref.at[pl.ds(off, sz)],
                dst_ref=recv_x_ref.at[pl.ds(off, sz)],
                send_sem=send_sems_x.at[t],
                recv_sem=recv_sems_x.at[t],
                device_id=x_nbr,
                device_id_type=pl.DeviceIdType.MESH,
            )
            rdma.start()
            x_rdmas.append(rdma)
            out_ref[pl.ds(row_me + off, sz), :] = (
                local_ref[pl.ds(row_me + off, sz), :]
                + recv_y_ref[pl.ds(off, sz), :]
            )

        for t, (off, sz) in enumerate(zip(OFFS, TILES)):
            x_rdmas[t].wait_recv()
            out_ref[pl.ds(row_other + off, sz), :] = (
                local_ref[pl.ds(row_other + off, sz), :]
                + recv_x_ref[pl.ds(off, sz), :]
            )

        for t in range(T):
            y_rdmas[t].wait_send()
            x_rdmas[t].wait_send()

    return pl.pallas_call(
        body,
        out_shape=jax.ShapeDtypeStruct((M, N_HALF), jnp.float32),
        in_specs=[pl.BlockSpec(memory_space=pl.ANY)],
        out_specs=pl.BlockSpec(memory_space=pltpu.VMEM),
        scratch_shapes=[
            pltpu.VMEM((M, N_HALF), jnp.float32),
            pltpu.VMEM((M_HALF, N_HALF), jnp.float32),
            pltpu.VMEM((M_HALF, N_HALF), jnp.float32),
            pltpu.SemaphoreType.DMA,
            pltpu.SemaphoreType.DMA((T,)),
            pltpu.SemaphoreType.DMA((T,)),
            pltpu.SemaphoreType.DMA((T,)),
            pltpu.SemaphoreType.DMA((T,)),
        ],
        compiler_params=pltpu.CompilerParams(collective_id=0),
    )(x)
